# baseline (speedup 1.0000x reference)
import functools

import jax
import jax.numpy as jnp
from jax import lax
from jax.experimental import pallas as pl
from jax.experimental.pallas import tpu as pltpu
from jax.experimental.pallas import tpu_sc as plsc

BATCH = 4096
HIST = 200
DIM = 64
HPAD = 208
HALF = HPAD // 2
NC = 2
NS = 16
NW = NC * NS
BPW = BATCH // NW
LANES = 16
NV = DIM // LANES


def _encoder_body(x_hbm, table_hbm, out_hbm, idx_v, rows_v, out_v, sem0, sem1):
    wid = lax.axis_index("s") * NC + lax.axis_index("c")
    base = wid * BPW

    pltpu.sync_copy(x_hbm.at[pl.ds(base, BPW)], idx_v)

    sems = (sem0, sem1)

    def start_gather(b, buf):
        c0 = pltpu.async_copy(
            table_hbm.at[idx_v.at[b, 0]], rows_v.at[buf, pl.ds(0, HALF)],
            sems[buf])
        c1 = pltpu.async_copy(
            table_hbm.at[idx_v.at[b, 1]], rows_v.at[buf, pl.ds(HALF, HALF)],
            sems[buf])
        return c0, c1

    def drain(buf):
        pltpu.make_async_copy(
            table_hbm.at[idx_v.at[0, 0]], rows_v.at[buf, pl.ds(0, HALF)],
            sems[buf]).wait()
        pltpu.make_async_copy(
            table_hbm.at[idx_v.at[0, 1]], rows_v.at[buf, pl.ds(HALF, HALF)],
            sems[buf]).wait()

    def accumulate(b, buf):
        zero = jnp.zeros((LANES,), jnp.float32)

        def body(j, acc):
            return tuple(
                acc[k] + rows_v[buf, j, pl.ds(k * LANES, LANES)]
                for k in range(NV))

        acc = lax.fori_loop(0, HIST, body, (zero,) * NV)
        one = jnp.full((LANES,), 1.0, jnp.float32)
        for k in range(NV):
            out_v[b, pl.ds(k * LANES, LANES)] = jnp.where(
                acc[k] > 0.0, one, -one)

    start_gather(0, 0)

    def outer(i, _):
        b0 = 2 * i
        start_gather(b0 + 1, 1)
        drain(0)
        accumulate(b0, 0)

        @pl.when(b0 + 2 < BPW)
        def _():
            start_gather(b0 + 2, 0)

        drain(1)
        accumulate(b0 + 1, 1)
        return 0

    lax.fori_loop(0, BPW // 2, outer, 0)

    pltpu.sync_copy(out_v, out_hbm.at[pl.ds(base, BPW)])


@jax.jit
def _encoder(x3, embed_weight):
    mesh = plsc.VectorSubcoreMesh(
        core_axis_name="c", subcore_axis_name="s", num_cores=NC,
        num_subcores=NS)
    return pl.kernel(
        _encoder_body,
        out_type=jax.ShapeDtypeStruct((BATCH, DIM), jnp.float32),
        mesh=mesh,
        scratch_types=[
            pltpu.VMEM((BPW, 2, HALF), jnp.int32),
            pltpu.VMEM((2, HPAD, DIM), jnp.float32),
            pltpu.VMEM((BPW, DIM), jnp.float32),
            pltpu.SemaphoreType.DMA,
            pltpu.SemaphoreType.DMA,
        ],
        compiler_params=pltpu.CompilerParams(use_tc_tiling_on_sc=False),
    )(x3, embed_weight)


def kernel(x, embed_weight):
    x3 = jnp.pad(x, ((0, 0), (0, HPAD - HIST))).reshape(BATCH, 2, HALF)
    return _encoder(x3, embed_weight)
